# baseline (speedup 1.0000x reference)
import functools

import jax
import jax.numpy as jnp
from jax import lax
from jax.experimental import pallas as pl
from jax.experimental.pallas import tpu as pltpu
from jax.experimental.pallas import tpu_sc as plsc

VOCAB = 100000
EMBED = 16
BATCH = 1024

ROW_BLK = 32
NSTEPS = BATCH // ROW_BLK
NSPLIT = 4
RCH = ROW_BLK // NSPLIT


def _sc_gather(emb_table, x):
    info = plsc.get_sparse_core_info()
    nc, ns = info.num_cores, info.num_subcores
    nw = nc * ns
    b_per_w = BATCH // nw

    mesh = plsc.VectorSubcoreMesh(core_axis_name="c", subcore_axis_name="s")

    @functools.partial(
        pl.kernel,
        mesh=mesh,
        out_type=jax.ShapeDtypeStruct((BATCH, EMBED), jnp.float32),
        scratch_types=[
            pltpu.VMEM((b_per_w,), jnp.int32),
            pltpu.VMEM((b_per_w, EMBED), jnp.float32),
            pltpu.SemaphoreType.DMA,
        ],
        compiler_params=pltpu.CompilerParams(use_tc_tiling_on_sc=False),
    )
    def gather_kernel(table_hbm, idx_hbm, out_hbm, idx_v, rows_v, sem):
        wid = lax.axis_index("s") * nc + lax.axis_index("c")
        base = wid * b_per_w
        pltpu.sync_copy(idx_hbm.at[pl.ds(base, b_per_w)], idx_v)
        pltpu.async_copy(table_hbm.at[idx_v], rows_v, sem).wait()
        pltpu.sync_copy(rows_v, out_hbm.at[pl.ds(base, b_per_w)])

    return gather_kernel(emb_table, x)


def _mm_body(e_ref, wt_ref, b_ref, out_hbm, acc, sems):
    i = pl.program_id(0)
    slot = lax.rem(i, 2)

    @pl.when(i >= 2)
    def _wait_prev():
        for s in range(NSPLIT):
            pltpu.make_async_copy(
                acc.at[slot, pl.ds(s * RCH, RCH), :],
                out_hbm.at[pl.ds((i - 2) * ROW_BLK + s * RCH, RCH), :],
                sems.at[slot, s],
            ).wait()

    acc[slot] = (
        lax.dot_general(
            e_ref[...],
            wt_ref[...],
            (((1,), (0,)), ((), ())),
            preferred_element_type=jnp.float32,
        )
        + b_ref[...]
    )

    for s in range(NSPLIT):
        pltpu.make_async_copy(
            acc.at[slot, pl.ds(s * RCH, RCH), :],
            out_hbm.at[pl.ds(i * ROW_BLK + s * RCH, RCH), :],
            sems.at[slot, s],
        ).start()

    @pl.when(i == NSTEPS - 1)
    def _drain():
        prev = lax.rem(i - 1, 2)
        for s in range(NSPLIT):
            pltpu.make_async_copy(
                acc.at[prev, pl.ds(s * RCH, RCH), :],
                out_hbm.at[pl.ds((i - 1) * ROW_BLK + s * RCH, RCH), :],
                sems.at[prev, s],
            ).wait()
            pltpu.make_async_copy(
                acc.at[slot, pl.ds(s * RCH, RCH), :],
                out_hbm.at[pl.ds(i * ROW_BLK + s * RCH, RCH), :],
                sems.at[slot, s],
            ).wait()


def _tc_project(e, Wt, b2):
    return pl.pallas_call(
        _mm_body,
        grid=(NSTEPS,),
        in_specs=[
            pl.BlockSpec((ROW_BLK, EMBED), lambda i: (i, 0)),
            pl.BlockSpec((EMBED, VOCAB), lambda i: (0, 0)),
            pl.BlockSpec((1, VOCAB), lambda i: (0, 0)),
        ],
        out_specs=pl.BlockSpec(memory_space=pl.ANY),
        out_shape=jax.ShapeDtypeStruct((BATCH, VOCAB), jnp.float32),
        scratch_shapes=[
            pltpu.VMEM((2, ROW_BLK, VOCAB), jnp.float32),
            pltpu.SemaphoreType.DMA((2, NSPLIT)),
        ],
        compiler_params=pltpu.CompilerParams(
            dimension_semantics=("arbitrary",),
        ),
    )(e, Wt, b2)


@jax.jit
def kernel(x, emb_table, W, b):
    e = jnp.take(emb_table, x, axis=0)
    return _tc_project(e, W.T, b.reshape(1, VOCAB))


# speedup vs baseline: 1.0038x; 1.0038x over previous
import functools

import jax
import jax.numpy as jnp
from jax import lax
from jax.experimental import pallas as pl
from jax.experimental.pallas import tpu as pltpu
from jax.experimental.pallas import tpu_sc as plsc

VOCAB = 100000
EMBED = 16
BATCH = 1024

ROW_BLK = 32
NSTEPS = BATCH // ROW_BLK
NSPLIT = 4
RCH = ROW_BLK // NSPLIT


def _sc_gather(emb_table, x):
    info = plsc.get_sparse_core_info()
    nc, ns = info.num_cores, info.num_subcores
    nw = nc * ns
    b_per_w = BATCH // nw

    mesh = plsc.VectorSubcoreMesh(core_axis_name="c", subcore_axis_name="s")

    @functools.partial(
        pl.kernel,
        mesh=mesh,
        out_type=jax.ShapeDtypeStruct((BATCH, EMBED), jnp.float32),
        scratch_types=[
            pltpu.VMEM((b_per_w,), jnp.int32),
            pltpu.VMEM((b_per_w, EMBED), jnp.float32),
            pltpu.SemaphoreType.DMA,
        ],
        compiler_params=pltpu.CompilerParams(use_tc_tiling_on_sc=False),
    )
    def gather_kernel(table_hbm, idx_hbm, out_hbm, idx_v, rows_v, sem):
        wid = lax.axis_index("s") * nc + lax.axis_index("c")
        base = wid * b_per_w
        pltpu.sync_copy(idx_hbm.at[pl.ds(base, b_per_w)], idx_v)
        pltpu.async_copy(table_hbm.at[idx_v], rows_v, sem).wait()
        pltpu.sync_copy(rows_v, out_hbm.at[pl.ds(base, b_per_w)])

    return gather_kernel(emb_table, x)


def _mm_body(e_ref, wt_ref, b_ref, out_hbm, acc, sems):
    i = pl.program_id(0)
    slot = lax.rem(i, 2)

    @pl.when(i >= 2)
    def _wait_prev():
        for s in range(NSPLIT):
            pltpu.make_async_copy(
                acc.at[slot, pl.ds(s * RCH, RCH), :],
                out_hbm.at[pl.ds((i - 2) * ROW_BLK + s * RCH, RCH), :],
                sems.at[slot, s],
            ).wait()

    acc[slot] = jnp.broadcast_to(b_ref[...], (ROW_BLK, VOCAB))

    for s in range(NSPLIT):
        pltpu.make_async_copy(
            acc.at[slot, pl.ds(s * RCH, RCH), :],
            out_hbm.at[pl.ds(i * ROW_BLK + s * RCH, RCH), :],
            sems.at[slot, s],
        ).start()

    @pl.when(i == NSTEPS - 1)
    def _drain():
        prev = lax.rem(i - 1, 2)
        for s in range(NSPLIT):
            pltpu.make_async_copy(
                acc.at[prev, pl.ds(s * RCH, RCH), :],
                out_hbm.at[pl.ds((i - 1) * ROW_BLK + s * RCH, RCH), :],
                sems.at[prev, s],
            ).wait()
            pltpu.make_async_copy(
                acc.at[slot, pl.ds(s * RCH, RCH), :],
                out_hbm.at[pl.ds(i * ROW_BLK + s * RCH, RCH), :],
                sems.at[slot, s],
            ).wait()


def _tc_project(e, Wt, b2):
    return pl.pallas_call(
        _mm_body,
        grid=(NSTEPS,),
        in_specs=[
            pl.BlockSpec((ROW_BLK, EMBED), lambda i: (i, 0)),
            pl.BlockSpec((EMBED, VOCAB), lambda i: (0, 0)),
            pl.BlockSpec((1, VOCAB), lambda i: (0, 0)),
        ],
        out_specs=pl.BlockSpec(memory_space=pl.ANY),
        out_shape=jax.ShapeDtypeStruct((BATCH, VOCAB), jnp.float32),
        scratch_shapes=[
            pltpu.VMEM((2, ROW_BLK, VOCAB), jnp.float32),
            pltpu.SemaphoreType.DMA((2, NSPLIT)),
        ],
        compiler_params=pltpu.CompilerParams(
            dimension_semantics=("arbitrary",),
        ),
    )(e, Wt, b2)


@jax.jit
def kernel(x, emb_table, W, b):
    e = jnp.take(emb_table, x, axis=0)
    return _tc_project(e, W.T, b.reshape(1, VOCAB))


# speedup vs baseline: 2.3907x; 2.3816x over previous
import functools

import jax
import jax.numpy as jnp
from jax import lax
from jax.experimental import pallas as pl
from jax.experimental.pallas import tpu as pltpu
from jax.experimental.pallas import tpu_sc as plsc

VOCAB = 100000
EMBED = 16
BATCH = 1024

V_BLK = 4096


def _sc_gather(emb_table, x):
    info = plsc.get_sparse_core_info()
    nc, ns = info.num_cores, info.num_subcores
    nw = nc * ns
    b_per_w = BATCH // nw

    mesh = plsc.VectorSubcoreMesh(core_axis_name="c", subcore_axis_name="s")

    @functools.partial(
        pl.kernel,
        mesh=mesh,
        out_type=jax.ShapeDtypeStruct((BATCH, EMBED), jnp.float32),
        scratch_types=[
            pltpu.VMEM((b_per_w,), jnp.int32),
            pltpu.VMEM((b_per_w, EMBED), jnp.float32),
            pltpu.SemaphoreType.DMA,
        ],
        compiler_params=pltpu.CompilerParams(use_tc_tiling_on_sc=False),
    )
    def gather_kernel(table_hbm, idx_hbm, out_hbm, idx_v, rows_v, sem):
        wid = lax.axis_index("s") * nc + lax.axis_index("c")
        base = wid * b_per_w
        pltpu.sync_copy(idx_hbm.at[pl.ds(base, b_per_w)], idx_v)
        pltpu.async_copy(table_hbm.at[idx_v], rows_v, sem).wait()
        pltpu.sync_copy(rows_v, out_hbm.at[pl.ds(base, b_per_w)])

    return gather_kernel(emb_table, x)


def _mm_body(e_ref, wt_ref, b_ref, o_ref):
    o_ref[...] = (
        lax.dot_general(
            wt_ref[...],
            e_ref[...],
            (((0,), (1,)), ((), ())),
            preferred_element_type=jnp.float32,
        )
        + b_ref[...]
    )


def _tc_project(e, Wt, b2):
    grid = (pl.cdiv(VOCAB, V_BLK),)
    return pl.pallas_call(
        _mm_body,
        grid=grid,
        in_specs=[
            pl.BlockSpec((BATCH, EMBED), lambda i: (0, 0)),
            pl.BlockSpec((EMBED, V_BLK), lambda i: (0, i)),
            pl.BlockSpec((V_BLK, 1), lambda i: (i, 0)),
        ],
        out_specs=pl.BlockSpec((V_BLK, BATCH), lambda i: (i, 0)),
        out_shape=jax.ShapeDtypeStruct((VOCAB, BATCH), jnp.float32),
        compiler_params=pltpu.CompilerParams(
            dimension_semantics=("parallel",),
        ),
    )(e, Wt, b2)


@jax.jit
def kernel(x, emb_table, W, b):
    e = jnp.take(emb_table, x, axis=0)
    logits_t = _tc_project(e, W.T, b.reshape(VOCAB, 1))
    return logits_t.T


# speedup vs baseline: 2.9740x; 1.2440x over previous
import functools

import jax
import jax.numpy as jnp
from jax import lax
from jax.experimental import pallas as pl
from jax.experimental.pallas import tpu as pltpu
from jax.experimental.pallas import tpu_sc as plsc

VOCAB = 100000
EMBED = 16
BATCH = 1024

V_BLK = 4096
K_AUG = EMBED + 1


def _sc_gather(emb_table, x):
    info = plsc.get_sparse_core_info()
    nc, ns = info.num_cores, info.num_subcores
    nw = nc * ns
    b_per_w = BATCH // nw

    mesh = plsc.VectorSubcoreMesh(core_axis_name="c", subcore_axis_name="s")

    @functools.partial(
        pl.kernel,
        mesh=mesh,
        out_type=jax.ShapeDtypeStruct((BATCH, EMBED), jnp.float32),
        scratch_types=[
            pltpu.VMEM((b_per_w,), jnp.int32),
            pltpu.VMEM((b_per_w, EMBED), jnp.float32),
            pltpu.SemaphoreType.DMA,
        ],
        compiler_params=pltpu.CompilerParams(use_tc_tiling_on_sc=False),
    )
    def gather_kernel(table_hbm, idx_hbm, out_hbm, idx_v, rows_v, sem):
        wid = lax.axis_index("s") * nc + lax.axis_index("c")
        base = wid * b_per_w
        pltpu.sync_copy(idx_hbm.at[pl.ds(base, b_per_w)], idx_v)
        pltpu.async_copy(table_hbm.at[idx_v], rows_v, sem).wait()
        pltpu.sync_copy(rows_v, out_hbm.at[pl.ds(base, b_per_w)])

    return gather_kernel(emb_table, x)


def _mm_body(e_ref, wt_ref, o_ref):
    o_ref[...] = lax.dot_general(
        wt_ref[...],
        e_ref[...],
        (((0,), (1,)), ((), ())),
        preferred_element_type=jnp.float32,
    )


def _tc_project(e_aug, Wtb):
    grid = (pl.cdiv(VOCAB, V_BLK),)
    return pl.pallas_call(
        _mm_body,
        grid=grid,
        in_specs=[
            pl.BlockSpec((BATCH, K_AUG), lambda i: (0, 0)),
            pl.BlockSpec((K_AUG, V_BLK), lambda i: (0, i)),
        ],
        out_specs=pl.BlockSpec((V_BLK, BATCH), lambda i: (i, 0)),
        out_shape=jax.ShapeDtypeStruct((VOCAB, BATCH), jnp.float32),
        compiler_params=pltpu.CompilerParams(
            dimension_semantics=("parallel",),
        ),
    )(e_aug, Wtb)


@jax.jit
def kernel(x, emb_table, W, b):
    e = jnp.take(emb_table, x, axis=0)
    e_aug = jnp.concatenate([e, jnp.ones((BATCH, 1), jnp.float32)], axis=1)
    Wtb = jnp.concatenate([W.T, b[None, :]], axis=0)
    logits_t = _tc_project(e_aug, Wtb)
    return logits_t.T
